# Initial kernel scaffold; baseline (speedup 1.0000x reference)
import jax
import jax.numpy as jnp
from jax.experimental import pallas as pl


def kernel(pos, zones_ids, W0, b0, W1, b1, A0, a0, A1, a1, C0, c0, C1, c1, G0, g0, G1, g1):
    raise NotImplementedError("write your pallas kernel here")


# speedup vs baseline: 5.9638x; 5.9638x over previous
import functools

import jax
import jax.numpy as jnp
from jax.experimental import pallas as pl

_INT_INF = 0x7F800000
_KMAX = 128


def _fps_stage(px, py, n_sample):
    b, np_ = px.shape
    j_n = jax.lax.broadcasted_iota(jnp.int32, (b, np_), 1)
    j_s = jax.lax.broadcasted_iota(jnp.int32, (b, n_sample), 1)

    def gather_col(x, sel):
        return jnp.sum(jnp.where(j_n == sel, x, jnp.float32(0.0)), axis=1,
                       keepdims=True)

    def body(i, st):
        sel, dmin, ox, oy = st
        lx = gather_col(px, sel)
        ly = gather_col(py, sel)
        ox = jnp.where(j_s == i - 1, lx, ox)
        oy = jnp.where(j_s == i - 1, ly, oy)
        d = (px - lx) ** 2 + (py - ly) ** 2
        dmin = jnp.minimum(dmin, d)
        m = jnp.max(dmin, axis=1, keepdims=True)
        nxt = jnp.min(jnp.where(dmin == m, j_n, jnp.int32(np_)), axis=1,
                      keepdims=True)
        return nxt, dmin, ox, oy

    init = (jnp.zeros((b, 1), jnp.int32),
            jnp.full((b, np_), jnp.inf, jnp.float32),
            jnp.zeros((b, n_sample), jnp.float32),
            jnp.zeros((b, n_sample), jnp.float32))
    sel, _, ox, oy = jax.lax.fori_loop(1, n_sample, body, init)
    lx = gather_col(px, sel)
    ly = gather_col(py, sel)
    ox = jnp.where(j_s == n_sample - 1, lx, ox)
    oy = jnp.where(j_s == n_sample - 1, ly, oy)
    return ox, oy


def _fps_body(px_ref, py_ref, p1x_ref, p1y_ref, p2x_ref, p2y_ref, *, s1, s2):
    px = px_ref[...]
    py = py_ref[...]
    p1x, p1y = _fps_stage(px, py, s1)
    p2x, p2y = _fps_stage(p1x, p1y, s2)
    p1x_ref[...] = p1x
    p1y_ref[...] = p1y
    p2x_ref[...] = p2x
    p2y_ref[...] = p2y


def _dot(a, b):
    return jnp.dot(a, b, precision=jax.lax.Precision.HIGHEST,
                   preferred_element_type=jnp.float32)


def _feat_body(p_ref, z_ref, w0_ref, b0_ref, w1_ref, b1_ref, a0a_ref, a0b_ref,
               a0c_ref, a0_ref, local_ref, t1_ref):
    p = p_ref[...]
    z = z_ref[...]
    h = jnp.tanh(_dot(p, w0_ref[...]) + b0_ref[...])
    loc = jnp.tanh(_dot(h, w1_ref[...]) + b1_ref[...])
    local_ref[...] = loc
    t1_ref[...] = (_dot(loc, a0a_ref[...]) + _dot(z, a0b_ref[...])
                   + _dot(p, a0c_ref[...]) + a0_ref[...])


def _select_mask_t(d2t, r2, kmax):
    s = d2t.shape[1]
    inrad = d2t <= jnp.float32(r2)
    u = jax.lax.bitcast_convert_type(d2t, jnp.int32)
    um = jnp.where(inrad, u, jnp.int32(_INT_INF))

    def bs(_, st):
        lo, hi = st
        mid = lo + (hi - lo) // 2
        cnt = jnp.sum((um <= mid).astype(jnp.int32), axis=0, keepdims=True)
        ge = cnt >= kmax
        return jnp.where(ge, lo, mid), jnp.where(ge, mid, hi)

    lo0 = jnp.full((1, s), -1, jnp.int32)
    hi0 = jnp.full((1, s), _INT_INF, jnp.int32)
    _, hi = jax.lax.fori_loop(0, 32, bs, (lo0, hi0))
    return jnp.logical_and(inrad, um <= hi)


def _conv_max(t, ucen, maskt, w2, b2):
    s = ucen.shape[0]
    np_ = t.shape[0]
    pre = jnp.concatenate([t - ucen[i:i + 1, :] for i in range(s)], axis=0)
    h2 = _dot(jnp.tanh(pre), w2) + b2
    mcol = jnp.concatenate([maskt[:, i:i + 1] for i in range(s)], axis=0)
    h2 = jnp.where(mcol, h2, -jnp.inf)
    return jnp.concatenate(
        [jnp.max(h2[i * np_:(i + 1) * np_, :], axis=0, keepdims=True)
         for i in range(s)], axis=0)


def _sa1_body(t_ref, pxc_ref, pyc_ref, cxr_ref, cyr_ref, cxc_ref, cyc_ref,
              wc_ref, w2_ref, b2_ref, out_ref, *, r2, kmax):
    t = t_ref[0]
    pxc = pxc_ref[0]
    pyc = pyc_ref[0]
    cxr = cxr_ref[0, 0]
    cyr = cyr_ref[0, 0]
    cxc = cxc_ref[0]
    cyc = cyc_ref[0]
    d2t = (cxr - pxc) ** 2 + (cyr - pyc) ** 2
    maskt = _select_mask_t(d2t, r2, kmax)
    ucen = cxc * wc_ref[0:1, :] + cyc * wc_ref[1:2, :]
    out_ref[...] = _conv_max(t, ucen, maskt, w2_ref[...], b2_ref[...])[None]


def _sa2_body(x1_ref, p1xc_ref, p1yc_ref, p2xr_ref, p2yr_ref, p2xc_ref,
              p2yc_ref, c0a_ref, c0c_ref, c0_ref, c1_ref, c1b_ref, out_ref,
              *, r2, kmax):
    x1 = x1_ref[0]
    p1xc = p1xc_ref[0]
    p1yc = p1yc_ref[0]
    p2xr = p2xr_ref[0]
    p2yr = p2yr_ref[0]
    p2xc = p2xc_ref[0]
    p2yc = p2yc_ref[0]
    c0c = c0c_ref[...]
    t2 = (_dot(x1, c0a_ref[...]) + p1xc * c0c[0:1, :] + p1yc * c0c[1:2, :]
          + c0_ref[...])
    d2t = (p2xr - p1xc) ** 2 + (p2yr - p1yc) ** 2
    maskt = _select_mask_t(d2t, r2, kmax)
    ucen = p2xc * c0c[0:1, :] + p2yc * c0c[1:2, :]
    out_ref[...] = _conv_max(t2, ucen, maskt, c1_ref[...],
                             c1b_ref[...])[None]


def _glob_body(x2_ref, p2x_ref, p2y_ref, g0a_ref, g0c_ref, g0_ref, g1_ref,
               g1b_ref, out_ref, *, b, s2):
    x2 = x2_ref[...]
    p2x = p2x_ref[...]
    p2y = p2y_ref[...]
    g0c = g0c_ref[...]
    g = jnp.tanh(_dot(x2, g0a_ref[...]) + p2x * g0c[0:1, :]
                 + p2y * g0c[1:2, :] + g0_ref[...])
    gg = _dot(g, g1_ref[...]) + g1b_ref[...]
    out_ref[...] = jnp.concatenate(
        [jnp.max(gg[i * s2:(i + 1) * s2, :], axis=0, keepdims=True)
         for i in range(b)], axis=0)


def kernel(pos, zones_ids, W0, b0, W1, b1, A0, a0, A1, a1, C0, c0, C1, c1,
           G0, g0, G1, g1):
    f32 = jnp.float32
    bb, nn, _ = pos.shape
    s1 = nn // 4
    s2 = s1 // 4
    st1 = 16
    r1sq = 0.2 * 0.2
    r2sq = 0.4 * 0.4

    posx = pos[:, :, 0]
    posy = pos[:, :, 1]

    p1x, p1y, p2x, p2y = pl.pallas_call(
        functools.partial(_fps_body, s1=s1, s2=s2),
        out_shape=(jax.ShapeDtypeStruct((bb, s1), f32),
                   jax.ShapeDtypeStruct((bb, s1), f32),
                   jax.ShapeDtypeStruct((bb, s2), f32),
                   jax.ShapeDtypeStruct((bb, s2), f32)),
    )(posx, posy)

    rt = 1024
    local2, t1f = pl.pallas_call(
        _feat_body,
        grid=(bb * nn // rt,),
        in_specs=[
            pl.BlockSpec((rt, 2), lambda r: (r, 0)),
            pl.BlockSpec((rt, 8), lambda r: (r, 0)),
            pl.BlockSpec((2, 32), lambda r: (0, 0)),
            pl.BlockSpec((1, 32), lambda r: (0, 0)),
            pl.BlockSpec((32, 32), lambda r: (0, 0)),
            pl.BlockSpec((1, 32), lambda r: (0, 0)),
            pl.BlockSpec((32, 64), lambda r: (0, 0)),
            pl.BlockSpec((8, 64), lambda r: (0, 0)),
            pl.BlockSpec((2, 64), lambda r: (0, 0)),
            pl.BlockSpec((1, 64), lambda r: (0, 0)),
        ],
        out_specs=(pl.BlockSpec((rt, 32), lambda r: (r, 0)),
                   pl.BlockSpec((rt, 64), lambda r: (r, 0))),
        out_shape=(jax.ShapeDtypeStruct((bb * nn, 32), f32),
                   jax.ShapeDtypeStruct((bb * nn, 64), f32)),
    )(pos.reshape(bb * nn, 2), zones_ids.reshape(bb * nn, 8),
      W0, b0.reshape(1, -1), W1, b1.reshape(1, -1),
      A0[0:32], A0[32:40], A0[40:42], a0.reshape(1, -1))

    t1 = t1f.reshape(bb, nn, 64)
    x1 = pl.pallas_call(
        functools.partial(_sa1_body, r2=r1sq, kmax=_KMAX),
        grid=(bb, s1 // st1),
        in_specs=[
            pl.BlockSpec((1, nn, 64), lambda b, s: (b, 0, 0)),
            pl.BlockSpec((1, nn, 1), lambda b, s: (b, 0, 0)),
            pl.BlockSpec((1, nn, 1), lambda b, s: (b, 0, 0)),
            pl.BlockSpec((1, 1, 1, st1), lambda b, s: (b, s, 0, 0)),
            pl.BlockSpec((1, 1, 1, st1), lambda b, s: (b, s, 0, 0)),
            pl.BlockSpec((1, st1, 1), lambda b, s: (b, s, 0)),
            pl.BlockSpec((1, st1, 1), lambda b, s: (b, s, 0)),
            pl.BlockSpec((2, 64), lambda b, s: (0, 0)),
            pl.BlockSpec((64, 64), lambda b, s: (0, 0)),
            pl.BlockSpec((1, 64), lambda b, s: (0, 0)),
        ],
        out_specs=pl.BlockSpec((1, st1, 64), lambda b, s: (b, s, 0)),
        out_shape=jax.ShapeDtypeStruct((bb, s1, 64), f32),
    )(t1, posx[:, :, None], posy[:, :, None],
      p1x.reshape(bb, s1 // st1, 1, st1), p1y.reshape(bb, s1 // st1, 1, st1),
      p1x[:, :, None], p1y[:, :, None],
      A0[40:42], A1, a1.reshape(1, -1))

    x2 = pl.pallas_call(
        functools.partial(_sa2_body, r2=r2sq, kmax=_KMAX),
        grid=(bb,),
        in_specs=[
            pl.BlockSpec((1, s1, 64), lambda b: (b, 0, 0)),
            pl.BlockSpec((1, s1, 1), lambda b: (b, 0, 0)),
            pl.BlockSpec((1, s1, 1), lambda b: (b, 0, 0)),
            pl.BlockSpec((1, 1, s2), lambda b: (b, 0, 0)),
            pl.BlockSpec((1, 1, s2), lambda b: (b, 0, 0)),
            pl.BlockSpec((1, s2, 1), lambda b: (b, 0, 0)),
            pl.BlockSpec((1, s2, 1), lambda b: (b, 0, 0)),
            pl.BlockSpec((64, 128), lambda b: (0, 0)),
            pl.BlockSpec((2, 128), lambda b: (0, 0)),
            pl.BlockSpec((1, 128), lambda b: (0, 0)),
            pl.BlockSpec((128, 128), lambda b: (0, 0)),
            pl.BlockSpec((1, 128), lambda b: (0, 0)),
        ],
        out_specs=pl.BlockSpec((1, s2, 128), lambda b: (b, 0, 0)),
        out_shape=jax.ShapeDtypeStruct((bb, s2, 128), f32),
    )(x1, p1x[:, :, None], p1y[:, :, None], p2x[:, None, :],
      p2y[:, None, :], p2x[:, :, None], p2y[:, :, None],
      C0[0:64], C0[64:66], c0.reshape(1, -1), C1, c1.reshape(1, -1))

    glob = pl.pallas_call(
        functools.partial(_glob_body, b=bb, s2=s2),
        out_shape=jax.ShapeDtypeStruct((bb, 256), f32),
    )(x2.reshape(bb * s2, 128), p2x.reshape(bb * s2, 1),
      p2y.reshape(bb * s2, 1), G0[0:128], G0[128:130], g0.reshape(1, -1),
      G1, g1.reshape(1, -1))

    return local2.reshape(bb, nn, 32), glob


# speedup vs baseline: 7.7900x; 1.3062x over previous
import functools

import jax
import jax.numpy as jnp
from jax.experimental import pallas as pl

_INT_INF = 0x7F800000
_KMAX = 128


def _fps_stage(px, py, n_sample):
    b, np_ = px.shape
    j_n = jax.lax.broadcasted_iota(jnp.int32, (b, np_), 1)
    j_s = jax.lax.broadcasted_iota(jnp.int32, (b, n_sample), 1)

    def gather_col(x, sel):
        return jnp.sum(jnp.where(j_n == sel, x, jnp.float32(0.0)), axis=1,
                       keepdims=True)

    def body(i, st):
        sel, dmin, ox, oy = st
        lx = gather_col(px, sel)
        ly = gather_col(py, sel)
        ox = jnp.where(j_s == i - 1, lx, ox)
        oy = jnp.where(j_s == i - 1, ly, oy)
        d = (px - lx) ** 2 + (py - ly) ** 2
        dmin = jnp.minimum(dmin, d)
        m = jnp.max(dmin, axis=1, keepdims=True)
        nxt = jnp.min(jnp.where(dmin == m, j_n, jnp.int32(np_)), axis=1,
                      keepdims=True)
        return nxt, dmin, ox, oy

    init = (jnp.zeros((b, 1), jnp.int32),
            jnp.full((b, np_), jnp.inf, jnp.float32),
            jnp.zeros((b, n_sample), jnp.float32),
            jnp.zeros((b, n_sample), jnp.float32))
    sel, _, ox, oy = jax.lax.fori_loop(1, n_sample, body, init)
    lx = gather_col(px, sel)
    ly = gather_col(py, sel)
    ox = jnp.where(j_s == n_sample - 1, lx, ox)
    oy = jnp.where(j_s == n_sample - 1, ly, oy)
    return ox, oy


def _fps_body(px_ref, py_ref, p1x_ref, p1y_ref, p2x_ref, p2y_ref, *, s1, s2):
    px = px_ref[...]
    py = py_ref[...]
    p1x, p1y = _fps_stage(px, py, s1)
    p2x, p2y = _fps_stage(p1x, p1y, s2)
    p1x_ref[...] = p1x
    p1y_ref[...] = p1y
    p2x_ref[...] = p2x
    p2y_ref[...] = p2y


def _dot(a, b):
    return jnp.dot(a, b, precision=jax.lax.Precision.HIGHEST,
                   preferred_element_type=jnp.float32)


def _feat_body(p_ref, z_ref, w0_ref, b0_ref, w1_ref, b1_ref, a0a_ref, a0b_ref,
               a0c_ref, a0_ref, local_ref, t1_ref):
    p = p_ref[...]
    z = z_ref[...]
    h = jnp.tanh(_dot(p, w0_ref[...]) + b0_ref[...])
    loc = jnp.tanh(_dot(h, w1_ref[...]) + b1_ref[...])
    local_ref[...] = loc
    t1_ref[...] = (_dot(loc, a0a_ref[...]) + _dot(z, a0b_ref[...])
                   + _dot(p, a0c_ref[...]) + a0_ref[...])


def _select_mask_t(d2t, r2, kmax):
    s = d2t.shape[1]
    inrad = d2t <= jnp.float32(r2)
    u = jax.lax.bitcast_convert_type(d2t, jnp.int32)
    um = jnp.where(inrad, u, jnp.int32(_INT_INF))

    def bs(_, st):
        lo, hi = st
        mid = lo + (hi - lo) // 2
        cnt = jnp.sum((um <= mid).astype(jnp.int32), axis=0, keepdims=True)
        ge = cnt >= kmax
        return jnp.where(ge, lo, mid), jnp.where(ge, mid, hi)

    lo0 = jnp.full((1, s), -1, jnp.int32)
    hi0 = jnp.full((1, s), _INT_INF, jnp.int32)
    _, hi = jax.lax.fori_loop(0, 32, bs, (lo0, hi0))
    return jnp.logical_and(inrad, um <= hi)


def _conv_max(t, ucen, pent, w2, b2):
    s = ucen.shape[0]
    np_ = t.shape[0]
    pre = jnp.concatenate([t - ucen[i:i + 1, :] for i in range(s)], axis=0)
    h2 = _dot(jnp.tanh(pre), w2) + b2
    pcol = jnp.concatenate([pent[:, i:i + 1] for i in range(s)], axis=0)
    h2 = h2 + pcol
    return jnp.concatenate(
        [jnp.max(h2[i * np_:(i + 1) * np_, :], axis=0, keepdims=True)
         for i in range(s)], axis=0)


def _pen_body(pxc_ref, pyc_ref, cxr_ref, cyr_ref, out_ref, *, r2, kmax, st):
    pxc = pxc_ref[0]
    pyc = pyc_ref[0]
    cxr = cxr_ref[0]
    cyr = cyr_ref[0]
    d2t = (cxr - pxc) ** 2 + (cyr - pyc) ** 2
    maskt = _select_mask_t(d2t, r2, kmax)
    pent = jnp.where(maskt, jnp.float32(0.0), -jnp.inf)
    s = d2t.shape[1]
    for k in range(s // st):
        out_ref[0, k] = pent[:, k * st:(k + 1) * st]


def _sa1_body(t_ref, pen_ref, cxc_ref, cyc_ref, wc_ref, w2_ref, b2_ref,
              out_ref):
    t = t_ref[0]
    pent = pen_ref[0, 0]
    cxc = cxc_ref[0]
    cyc = cyc_ref[0]
    ucen = cxc * wc_ref[0:1, :] + cyc * wc_ref[1:2, :]
    out_ref[...] = _conv_max(t, ucen, pent, w2_ref[...], b2_ref[...])[None]


def _sa2_body(x1_ref, p1xc_ref, p1yc_ref, p2xr_ref, p2yr_ref, p2xc_ref,
              p2yc_ref, c0a_ref, c0c_ref, c0_ref, c1_ref, c1b_ref, out_ref,
              *, r2, kmax):
    x1 = x1_ref[0]
    p1xc = p1xc_ref[0]
    p1yc = p1yc_ref[0]
    p2xr = p2xr_ref[0]
    p2yr = p2yr_ref[0]
    p2xc = p2xc_ref[0]
    p2yc = p2yc_ref[0]
    c0c = c0c_ref[...]
    t2 = (_dot(x1, c0a_ref[...]) + p1xc * c0c[0:1, :] + p1yc * c0c[1:2, :]
          + c0_ref[...])
    d2t = (p2xr - p1xc) ** 2 + (p2yr - p1yc) ** 2
    maskt = _select_mask_t(d2t, r2, kmax)
    pent = jnp.where(maskt, jnp.float32(0.0), -jnp.inf)
    ucen = p2xc * c0c[0:1, :] + p2yc * c0c[1:2, :]
    out_ref[...] = _conv_max(t2, ucen, pent, c1_ref[...],
                             c1b_ref[...])[None]


def _glob_body(x2_ref, p2x_ref, p2y_ref, g0a_ref, g0c_ref, g0_ref, g1_ref,
               g1b_ref, out_ref, *, b, s2):
    x2 = x2_ref[...]
    p2x = p2x_ref[...]
    p2y = p2y_ref[...]
    g0c = g0c_ref[...]
    g = jnp.tanh(_dot(x2, g0a_ref[...]) + p2x * g0c[0:1, :]
                 + p2y * g0c[1:2, :] + g0_ref[...])
    gg = _dot(g, g1_ref[...]) + g1b_ref[...]
    out_ref[...] = jnp.concatenate(
        [jnp.max(gg[i * s2:(i + 1) * s2, :], axis=0, keepdims=True)
         for i in range(b)], axis=0)


def kernel(pos, zones_ids, W0, b0, W1, b1, A0, a0, A1, a1, C0, c0, C1, c1,
           G0, g0, G1, g1):
    f32 = jnp.float32
    bb, nn, _ = pos.shape
    s1 = nn // 4
    s2 = s1 // 4
    st1 = 16
    r1sq = 0.2 * 0.2
    r2sq = 0.4 * 0.4

    posx = pos[:, :, 0]
    posy = pos[:, :, 1]

    p1x, p1y, p2x, p2y = pl.pallas_call(
        functools.partial(_fps_body, s1=s1, s2=s2),
        out_shape=(jax.ShapeDtypeStruct((bb, s1), f32),
                   jax.ShapeDtypeStruct((bb, s1), f32),
                   jax.ShapeDtypeStruct((bb, s2), f32),
                   jax.ShapeDtypeStruct((bb, s2), f32)),
    )(posx, posy)

    rt = 1024
    local2, t1f = pl.pallas_call(
        _feat_body,
        grid=(bb * nn // rt,),
        in_specs=[
            pl.BlockSpec((rt, 2), lambda r: (r, 0)),
            pl.BlockSpec((rt, 8), lambda r: (r, 0)),
            pl.BlockSpec((2, 32), lambda r: (0, 0)),
            pl.BlockSpec((1, 32), lambda r: (0, 0)),
            pl.BlockSpec((32, 32), lambda r: (0, 0)),
            pl.BlockSpec((1, 32), lambda r: (0, 0)),
            pl.BlockSpec((32, 64), lambda r: (0, 0)),
            pl.BlockSpec((8, 64), lambda r: (0, 0)),
            pl.BlockSpec((2, 64), lambda r: (0, 0)),
            pl.BlockSpec((1, 64), lambda r: (0, 0)),
        ],
        out_specs=(pl.BlockSpec((rt, 32), lambda r: (r, 0)),
                   pl.BlockSpec((rt, 64), lambda r: (r, 0))),
        out_shape=(jax.ShapeDtypeStruct((bb * nn, 32), f32),
                   jax.ShapeDtypeStruct((bb * nn, 64), f32)),
    )(pos.reshape(bb * nn, 2), zones_ids.reshape(bb * nn, 8),
      W0, b0.reshape(1, -1), W1, b1.reshape(1, -1),
      A0[0:32], A0[32:40], A0[40:42], a0.reshape(1, -1))

    pen1 = pl.pallas_call(
        functools.partial(_pen_body, r2=r1sq, kmax=_KMAX, st=st1),
        grid=(bb,),
        in_specs=[
            pl.BlockSpec((1, nn, 1), lambda b: (b, 0, 0)),
            pl.BlockSpec((1, nn, 1), lambda b: (b, 0, 0)),
            pl.BlockSpec((1, 1, s1), lambda b: (b, 0, 0)),
            pl.BlockSpec((1, 1, s1), lambda b: (b, 0, 0)),
        ],
        out_specs=pl.BlockSpec((1, s1 // st1, nn, st1), lambda b: (b, 0, 0, 0)),
        out_shape=jax.ShapeDtypeStruct((bb, s1 // st1, nn, st1), f32),
    )(posx[:, :, None], posy[:, :, None], p1x[:, None, :], p1y[:, None, :])

    t1 = t1f.reshape(bb, nn, 64)
    x1 = pl.pallas_call(
        _sa1_body,
        grid=(bb, s1 // st1),
        in_specs=[
            pl.BlockSpec((1, nn, 64), lambda b, s: (b, 0, 0)),
            pl.BlockSpec((1, 1, nn, st1), lambda b, s: (b, s, 0, 0)),
            pl.BlockSpec((1, st1, 1), lambda b, s: (b, s, 0)),
            pl.BlockSpec((1, st1, 1), lambda b, s: (b, s, 0)),
            pl.BlockSpec((2, 64), lambda b, s: (0, 0)),
            pl.BlockSpec((64, 64), lambda b, s: (0, 0)),
            pl.BlockSpec((1, 64), lambda b, s: (0, 0)),
        ],
        out_specs=pl.BlockSpec((1, st1, 64), lambda b, s: (b, s, 0)),
        out_shape=jax.ShapeDtypeStruct((bb, s1, 64), f32),
    )(t1, pen1, p1x[:, :, None], p1y[:, :, None],
      A0[40:42], A1, a1.reshape(1, -1))

    x2 = pl.pallas_call(
        functools.partial(_sa2_body, r2=r2sq, kmax=_KMAX),
        grid=(bb,),
        in_specs=[
            pl.BlockSpec((1, s1, 64), lambda b: (b, 0, 0)),
            pl.BlockSpec((1, s1, 1), lambda b: (b, 0, 0)),
            pl.BlockSpec((1, s1, 1), lambda b: (b, 0, 0)),
            pl.BlockSpec((1, 1, s2), lambda b: (b, 0, 0)),
            pl.BlockSpec((1, 1, s2), lambda b: (b, 0, 0)),
            pl.BlockSpec((1, s2, 1), lambda b: (b, 0, 0)),
            pl.BlockSpec((1, s2, 1), lambda b: (b, 0, 0)),
            pl.BlockSpec((64, 128), lambda b: (0, 0)),
            pl.BlockSpec((2, 128), lambda b: (0, 0)),
            pl.BlockSpec((1, 128), lambda b: (0, 0)),
            pl.BlockSpec((128, 128), lambda b: (0, 0)),
            pl.BlockSpec((1, 128), lambda b: (0, 0)),
        ],
        out_specs=pl.BlockSpec((1, s2, 128), lambda b: (b, 0, 0)),
        out_shape=jax.ShapeDtypeStruct((bb, s2, 128), f32),
    )(x1, p1x[:, :, None], p1y[:, :, None], p2x[:, None, :],
      p2y[:, None, :], p2x[:, :, None], p2y[:, :, None],
      C0[0:64], C0[64:66], c0.reshape(1, -1), C1, c1.reshape(1, -1))

    glob = pl.pallas_call(
        functools.partial(_glob_body, b=bb, s2=s2),
        out_shape=jax.ShapeDtypeStruct((bb, 256), f32),
    )(x2.reshape(bb * s2, 128), p2x.reshape(bb * s2, 1),
      p2y.reshape(bb * s2, 1), G0[0:128], G0[128:130], g0.reshape(1, -1),
      G1, g1.reshape(1, -1))

    return local2.reshape(bb, nn, 32), glob


# speedup vs baseline: 7.7959x; 1.0008x over previous
import functools

import jax
import jax.numpy as jnp
from jax.experimental import pallas as pl
from jax.experimental.pallas import tpu as pltpu

_PAR1 = pltpu.CompilerParams(dimension_semantics=("parallel",))
_PAR2 = pltpu.CompilerParams(dimension_semantics=("parallel", "parallel"))

_INT_INF = 0x7F800000
_KMAX = 128


def _fps_stage(px, py, n_sample):
    b, np_ = px.shape
    j_n = jax.lax.broadcasted_iota(jnp.int32, (b, np_), 1)
    j_s = jax.lax.broadcasted_iota(jnp.int32, (b, n_sample), 1)

    def gather_col(x, sel):
        return jnp.sum(jnp.where(j_n == sel, x, jnp.float32(0.0)), axis=1,
                       keepdims=True)

    def body(i, st):
        sel, dmin, ox, oy = st
        lx = gather_col(px, sel)
        ly = gather_col(py, sel)
        ox = jnp.where(j_s == i - 1, lx, ox)
        oy = jnp.where(j_s == i - 1, ly, oy)
        d = (px - lx) ** 2 + (py - ly) ** 2
        dmin = jnp.minimum(dmin, d)
        m = jnp.max(dmin, axis=1, keepdims=True)
        nxt = jnp.min(jnp.where(dmin == m, j_n, jnp.int32(np_)), axis=1,
                      keepdims=True)
        return nxt, dmin, ox, oy

    init = (jnp.zeros((b, 1), jnp.int32),
            jnp.full((b, np_), jnp.inf, jnp.float32),
            jnp.zeros((b, n_sample), jnp.float32),
            jnp.zeros((b, n_sample), jnp.float32))
    sel, _, ox, oy = jax.lax.fori_loop(1, n_sample, body, init)
    lx = gather_col(px, sel)
    ly = gather_col(py, sel)
    ox = jnp.where(j_s == n_sample - 1, lx, ox)
    oy = jnp.where(j_s == n_sample - 1, ly, oy)
    return ox, oy


def _fps_body(px_ref, py_ref, p1x_ref, p1y_ref, p2x_ref, p2y_ref, *, s1, s2):
    px = px_ref[...]
    py = py_ref[...]
    p1x, p1y = _fps_stage(px, py, s1)
    p2x, p2y = _fps_stage(p1x, p1y, s2)
    p1x_ref[...] = p1x
    p1y_ref[...] = p1y
    p2x_ref[...] = p2x
    p2y_ref[...] = p2y


def _dot(a, b):
    return jnp.dot(a, b, precision=jax.lax.Precision.HIGHEST,
                   preferred_element_type=jnp.float32)


def _feat_body(p_ref, z_ref, w0_ref, b0_ref, w1_ref, b1_ref, a0a_ref, a0b_ref,
               a0c_ref, a0_ref, local_ref, t1_ref):
    p = p_ref[...]
    z = z_ref[...]
    h = jnp.tanh(_dot(p, w0_ref[...]) + b0_ref[...])
    loc = jnp.tanh(_dot(h, w1_ref[...]) + b1_ref[...])
    local_ref[...] = loc
    t1_ref[...] = (_dot(loc, a0a_ref[...]) + _dot(z, a0b_ref[...])
                   + _dot(p, a0c_ref[...]) + a0_ref[...])


def _select_mask_t(d2t, r2, kmax):
    s = d2t.shape[1]
    inrad = d2t <= jnp.float32(r2)
    u = jax.lax.bitcast_convert_type(d2t, jnp.int32)
    um = jnp.where(inrad, u, jnp.int32(_INT_INF))

    def bs(_, st):
        lo, hi = st
        mid = lo + (hi - lo) // 2
        cnt = jnp.sum((um <= mid).astype(jnp.int32), axis=0, keepdims=True)
        ge = cnt >= kmax
        return jnp.where(ge, lo, mid), jnp.where(ge, mid, hi)

    lo0 = jnp.full((1, s), -1, jnp.int32)
    hi0 = jnp.full((1, s), _INT_INF, jnp.int32)
    _, hi = jax.lax.fori_loop(0, 32, bs, (lo0, hi0))
    return jnp.logical_and(inrad, um <= hi)


def _conv_max(t, ucen, pent, w2, b2):
    s = ucen.shape[0]
    np_ = t.shape[0]
    pre = jnp.concatenate([t - ucen[i:i + 1, :] for i in range(s)], axis=0)
    h2 = _dot(jnp.tanh(pre), w2) + b2
    pcol = jnp.concatenate([pent[:, i:i + 1] for i in range(s)], axis=0)
    h2 = h2 + pcol
    return jnp.concatenate(
        [jnp.max(h2[i * np_:(i + 1) * np_, :], axis=0, keepdims=True)
         for i in range(s)], axis=0)


def _pen_body(pxc_ref, pyc_ref, cxr_ref, cyr_ref, out_ref, *, r2, kmax, st):
    pxc = pxc_ref[0]
    pyc = pyc_ref[0]
    cxr = cxr_ref[0]
    cyr = cyr_ref[0]
    d2t = (cxr - pxc) ** 2 + (cyr - pyc) ** 2
    maskt = _select_mask_t(d2t, r2, kmax)
    pent = jnp.where(maskt, jnp.float32(0.0), -jnp.inf)
    s = d2t.shape[1]
    for k in range(s // st):
        out_ref[0, k] = pent[:, k * st:(k + 1) * st]


def _sa1_body(t_ref, pen_ref, cxc_ref, cyc_ref, wc_ref, w2_ref, b2_ref,
              out_ref):
    t = t_ref[0]
    pent = pen_ref[0, 0]
    cxc = cxc_ref[0]
    cyc = cyc_ref[0]
    ucen = cxc * wc_ref[0:1, :] + cyc * wc_ref[1:2, :]
    out_ref[...] = _conv_max(t, ucen, pent, w2_ref[...], b2_ref[...])[None]


def _sa2_body(x1_ref, p1xc_ref, p1yc_ref, p2xr_ref, p2yr_ref, p2xc_ref,
              p2yc_ref, c0a_ref, c0c_ref, c0_ref, c1_ref, c1b_ref, out_ref,
              *, r2, kmax):
    x1 = x1_ref[0]
    p1xc = p1xc_ref[0]
    p1yc = p1yc_ref[0]
    p2xr = p2xr_ref[0]
    p2yr = p2yr_ref[0]
    p2xc = p2xc_ref[0]
    p2yc = p2yc_ref[0]
    c0c = c0c_ref[...]
    t2 = (_dot(x1, c0a_ref[...]) + p1xc * c0c[0:1, :] + p1yc * c0c[1:2, :]
          + c0_ref[...])
    d2t = (p2xr - p1xc) ** 2 + (p2yr - p1yc) ** 2
    maskt = _select_mask_t(d2t, r2, kmax)
    pent = jnp.where(maskt, jnp.float32(0.0), -jnp.inf)
    ucen = p2xc * c0c[0:1, :] + p2yc * c0c[1:2, :]
    out_ref[...] = _conv_max(t2, ucen, pent, c1_ref[...],
                             c1b_ref[...])[None]


def _glob_body(x2_ref, p2x_ref, p2y_ref, g0a_ref, g0c_ref, g0_ref, g1_ref,
               g1b_ref, out_ref, *, b, s2):
    x2 = x2_ref[...]
    p2x = p2x_ref[...]
    p2y = p2y_ref[...]
    g0c = g0c_ref[...]
    g = jnp.tanh(_dot(x2, g0a_ref[...]) + p2x * g0c[0:1, :]
                 + p2y * g0c[1:2, :] + g0_ref[...])
    gg = _dot(g, g1_ref[...]) + g1b_ref[...]
    out_ref[...] = jnp.concatenate(
        [jnp.max(gg[i * s2:(i + 1) * s2, :], axis=0, keepdims=True)
         for i in range(b)], axis=0)


def kernel(pos, zones_ids, W0, b0, W1, b1, A0, a0, A1, a1, C0, c0, C1, c1,
           G0, g0, G1, g1):
    f32 = jnp.float32
    bb, nn, _ = pos.shape
    s1 = nn // 4
    s2 = s1 // 4
    st1 = 16
    r1sq = 0.2 * 0.2
    r2sq = 0.4 * 0.4

    posx = pos[:, :, 0]
    posy = pos[:, :, 1]

    p1x, p1y, p2x, p2y = pl.pallas_call(
        functools.partial(_fps_body, s1=s1, s2=s2),
        out_shape=(jax.ShapeDtypeStruct((bb, s1), f32),
                   jax.ShapeDtypeStruct((bb, s1), f32),
                   jax.ShapeDtypeStruct((bb, s2), f32),
                   jax.ShapeDtypeStruct((bb, s2), f32)),
    )(posx, posy)

    rt = 1024
    local2, t1f = pl.pallas_call(
        _feat_body,
        grid=(bb * nn // rt,),
        in_specs=[
            pl.BlockSpec((rt, 2), lambda r: (r, 0)),
            pl.BlockSpec((rt, 8), lambda r: (r, 0)),
            pl.BlockSpec((2, 32), lambda r: (0, 0)),
            pl.BlockSpec((1, 32), lambda r: (0, 0)),
            pl.BlockSpec((32, 32), lambda r: (0, 0)),
            pl.BlockSpec((1, 32), lambda r: (0, 0)),
            pl.BlockSpec((32, 64), lambda r: (0, 0)),
            pl.BlockSpec((8, 64), lambda r: (0, 0)),
            pl.BlockSpec((2, 64), lambda r: (0, 0)),
            pl.BlockSpec((1, 64), lambda r: (0, 0)),
        ],
        out_specs=(pl.BlockSpec((rt, 32), lambda r: (r, 0)),
                   pl.BlockSpec((rt, 64), lambda r: (r, 0))),
        out_shape=(jax.ShapeDtypeStruct((bb * nn, 32), f32),
                   jax.ShapeDtypeStruct((bb * nn, 64), f32)),
        compiler_params=_PAR1,
    )(pos.reshape(bb * nn, 2), zones_ids.reshape(bb * nn, 8),
      W0, b0.reshape(1, -1), W1, b1.reshape(1, -1),
      A0[0:32], A0[32:40], A0[40:42], a0.reshape(1, -1))

    pen1 = pl.pallas_call(
        functools.partial(_pen_body, r2=r1sq, kmax=_KMAX, st=st1),
        grid=(bb,),
        in_specs=[
            pl.BlockSpec((1, nn, 1), lambda b: (b, 0, 0)),
            pl.BlockSpec((1, nn, 1), lambda b: (b, 0, 0)),
            pl.BlockSpec((1, 1, s1), lambda b: (b, 0, 0)),
            pl.BlockSpec((1, 1, s1), lambda b: (b, 0, 0)),
        ],
        out_specs=pl.BlockSpec((1, s1 // st1, nn, st1), lambda b: (b, 0, 0, 0)),
        out_shape=jax.ShapeDtypeStruct((bb, s1 // st1, nn, st1), f32),
        compiler_params=_PAR1,
    )(posx[:, :, None], posy[:, :, None], p1x[:, None, :], p1y[:, None, :])

    t1 = t1f.reshape(bb, nn, 64)
    x1 = pl.pallas_call(
        _sa1_body,
        grid=(bb, s1 // st1),
        in_specs=[
            pl.BlockSpec((1, nn, 64), lambda b, s: (b, 0, 0)),
            pl.BlockSpec((1, 1, nn, st1), lambda b, s: (b, s, 0, 0)),
            pl.BlockSpec((1, st1, 1), lambda b, s: (b, s, 0)),
            pl.BlockSpec((1, st1, 1), lambda b, s: (b, s, 0)),
            pl.BlockSpec((2, 64), lambda b, s: (0, 0)),
            pl.BlockSpec((64, 64), lambda b, s: (0, 0)),
            pl.BlockSpec((1, 64), lambda b, s: (0, 0)),
        ],
        out_specs=pl.BlockSpec((1, st1, 64), lambda b, s: (b, s, 0)),
        out_shape=jax.ShapeDtypeStruct((bb, s1, 64), f32),
        compiler_params=_PAR2,
    )(t1, pen1, p1x[:, :, None], p1y[:, :, None],
      A0[40:42], A1, a1.reshape(1, -1))

    x2 = pl.pallas_call(
        functools.partial(_sa2_body, r2=r2sq, kmax=_KMAX),
        grid=(bb,),
        in_specs=[
            pl.BlockSpec((1, s1, 64), lambda b: (b, 0, 0)),
            pl.BlockSpec((1, s1, 1), lambda b: (b, 0, 0)),
            pl.BlockSpec((1, s1, 1), lambda b: (b, 0, 0)),
            pl.BlockSpec((1, 1, s2), lambda b: (b, 0, 0)),
            pl.BlockSpec((1, 1, s2), lambda b: (b, 0, 0)),
            pl.BlockSpec((1, s2, 1), lambda b: (b, 0, 0)),
            pl.BlockSpec((1, s2, 1), lambda b: (b, 0, 0)),
            pl.BlockSpec((64, 128), lambda b: (0, 0)),
            pl.BlockSpec((2, 128), lambda b: (0, 0)),
            pl.BlockSpec((1, 128), lambda b: (0, 0)),
            pl.BlockSpec((128, 128), lambda b: (0, 0)),
            pl.BlockSpec((1, 128), lambda b: (0, 0)),
        ],
        out_specs=pl.BlockSpec((1, s2, 128), lambda b: (b, 0, 0)),
        out_shape=jax.ShapeDtypeStruct((bb, s2, 128), f32),
        compiler_params=_PAR1,
    )(x1, p1x[:, :, None], p1y[:, :, None], p2x[:, None, :],
      p2y[:, None, :], p2x[:, :, None], p2y[:, :, None],
      C0[0:64], C0[64:66], c0.reshape(1, -1), C1, c1.reshape(1, -1))

    glob = pl.pallas_call(
        functools.partial(_glob_body, b=bb, s2=s2),
        out_shape=jax.ShapeDtypeStruct((bb, 256), f32),
    )(x2.reshape(bb * s2, 128), p2x.reshape(bb * s2, 1),
      p2y.reshape(bb * s2, 1), G0[0:128], G0[128:130], g0.reshape(1, -1),
      G1, g1.reshape(1, -1))

    return local2.reshape(bb, nn, 32), glob


# speedup vs baseline: 11.5082x; 1.4762x over previous
import functools

import jax
import jax.numpy as jnp
from jax.experimental import pallas as pl
from jax.experimental.pallas import tpu as pltpu

_PAR1 = pltpu.CompilerParams(dimension_semantics=("parallel",))
_PAR2 = pltpu.CompilerParams(dimension_semantics=("parallel", "parallel"))

_INT_INF = 0x7F800000
_KMAX = 128


def _fps_stage(px, py, n_sample):
    b, np_ = px.shape
    j_n = jax.lax.broadcasted_iota(jnp.int32, (b, np_), 1)
    j_s = jax.lax.broadcasted_iota(jnp.int32, (b, n_sample), 1)

    def gather_col(x, sel):
        return jnp.sum(jnp.where(j_n == sel, x, jnp.float32(0.0)), axis=1,
                       keepdims=True)

    def body(i, st):
        sel, dmin, ox, oy = st
        lx = gather_col(px, sel)
        ly = gather_col(py, sel)
        ox = jnp.where(j_s == i - 1, lx, ox)
        oy = jnp.where(j_s == i - 1, ly, oy)
        d = (px - lx) ** 2 + (py - ly) ** 2
        dmin = jnp.minimum(dmin, d)
        m = jnp.max(dmin, axis=1, keepdims=True)
        nxt = jnp.min(jnp.where(dmin == m, j_n, jnp.int32(np_)), axis=1,
                      keepdims=True)
        return nxt, dmin, ox, oy

    init = (jnp.zeros((b, 1), jnp.int32),
            jnp.full((b, np_), jnp.inf, jnp.float32),
            jnp.zeros((b, n_sample), jnp.float32),
            jnp.zeros((b, n_sample), jnp.float32))
    sel, _, ox, oy = jax.lax.fori_loop(1, n_sample, body, init)
    lx = gather_col(px, sel)
    ly = gather_col(py, sel)
    ox = jnp.where(j_s == n_sample - 1, lx, ox)
    oy = jnp.where(j_s == n_sample - 1, ly, oy)
    return ox, oy


def _fps_body(px_ref, py_ref, p1x_ref, p1y_ref, p2x_ref, p2y_ref, *, s1, s2):
    px = px_ref[...]
    py = py_ref[...]
    p1x, p1y = _fps_stage(px, py, s1)
    p2x, p2y = _fps_stage(p1x, p1y, s2)
    p1x_ref[...] = p1x
    p1y_ref[...] = p1y
    p2x_ref[...] = p2x
    p2y_ref[...] = p2y


def _dot(a, b):
    return jnp.dot(a, b, precision=jax.lax.Precision.HIGHEST,
                   preferred_element_type=jnp.float32)


def _feat_body(p_ref, z_ref, w0_ref, b0_ref, w1_ref, b1_ref, a0a_ref, a0b_ref,
               a0c_ref, a0_ref, local_ref, t1_ref):
    p = p_ref[...]
    z = z_ref[...]
    h = jnp.tanh(_dot(p, w0_ref[...]) + b0_ref[...])
    loc = jnp.tanh(_dot(h, w1_ref[...]) + b1_ref[...])
    local_ref[...] = loc
    t1_ref[...] = (_dot(loc, a0a_ref[...]) + _dot(z, a0b_ref[...])
                   + _dot(p, a0c_ref[...]) + a0_ref[...])


def _select_mask_t(d2t, r2, kmax):
    s = d2t.shape[1]
    inrad = d2t <= jnp.float32(r2)
    u = jax.lax.bitcast_convert_type(d2t, jnp.int32)
    um = jnp.where(inrad, u, jnp.int32(_INT_INF))

    def bs(_, st):
        lo, hi = st
        mid = lo + (hi - lo) // 2
        cnt = jnp.sum((um <= mid).astype(jnp.int32), axis=0, keepdims=True)
        ge = cnt >= kmax
        return jnp.where(ge, lo, mid), jnp.where(ge, mid, hi)

    lo0 = jnp.full((1, s), -1, jnp.int32)
    hi0 = jnp.full((1, s), _INT_INF, jnp.int32)
    _, hi = jax.lax.fori_loop(0, 32, bs, (lo0, hi0))
    return jnp.logical_and(inrad, um <= hi)


def _conv_max(t, ucen, pent, w2, b2):
    s = ucen.shape[0]
    np_ = t.shape[0]
    pre = jnp.concatenate([t - ucen[i:i + 1, :] for i in range(s)], axis=0)
    h2 = jnp.dot(jnp.tanh(pre), w2,
                 preferred_element_type=jnp.float32) + b2
    pcol = jnp.concatenate([pent[:, i:i + 1] for i in range(s)], axis=0)
    h2 = h2 + pcol
    return jnp.concatenate(
        [jnp.max(h2[i * np_:(i + 1) * np_, :], axis=0, keepdims=True)
         for i in range(s)], axis=0)


def _pen_body(pxc_ref, pyc_ref, cxr_ref, cyr_ref, out_ref, *, r2, kmax, st):
    pxc = pxc_ref[0]
    pyc = pyc_ref[0]
    cxr = cxr_ref[0]
    cyr = cyr_ref[0]
    d2t = (cxr - pxc) ** 2 + (cyr - pyc) ** 2
    maskt = _select_mask_t(d2t, r2, kmax)
    pent = jnp.where(maskt, jnp.float32(0.0), -jnp.inf)
    s = d2t.shape[1]
    for k in range(s // st):
        out_ref[0, k] = pent[:, k * st:(k + 1) * st]


def _sa1_body(t_ref, pen_ref, cxc_ref, cyc_ref, wc_ref, w2_ref, b2_ref,
              out_ref):
    t = t_ref[0]
    pent = pen_ref[0, 0]
    cxc = cxc_ref[0]
    cyc = cyc_ref[0]
    ucen = cxc * wc_ref[0:1, :] + cyc * wc_ref[1:2, :]
    out_ref[...] = _conv_max(t, ucen, pent, w2_ref[...], b2_ref[...])[None]


def _sa2_body(x1_ref, p1xc_ref, p1yc_ref, p2xr_ref, p2yr_ref, p2xc_ref,
              p2yc_ref, c0a_ref, c0c_ref, c0_ref, c1_ref, c1b_ref, out_ref,
              *, r2, kmax):
    x1 = x1_ref[0]
    p1xc = p1xc_ref[0]
    p1yc = p1yc_ref[0]
    p2xr = p2xr_ref[0]
    p2yr = p2yr_ref[0]
    p2xc = p2xc_ref[0]
    p2yc = p2yc_ref[0]
    c0c = c0c_ref[...]
    t2 = (_dot(x1, c0a_ref[...]) + p1xc * c0c[0:1, :] + p1yc * c0c[1:2, :]
          + c0_ref[...])
    d2t = (p2xr - p1xc) ** 2 + (p2yr - p1yc) ** 2
    maskt = _select_mask_t(d2t, r2, kmax)
    pent = jnp.where(maskt, jnp.float32(0.0), -jnp.inf)
    ucen = p2xc * c0c[0:1, :] + p2yc * c0c[1:2, :]
    out_ref[...] = _conv_max(t2, ucen, pent, c1_ref[...],
                             c1b_ref[...])[None]


def _glob_body(x2_ref, p2x_ref, p2y_ref, g0a_ref, g0c_ref, g0_ref, g1_ref,
               g1b_ref, out_ref, *, b, s2):
    x2 = x2_ref[...]
    p2x = p2x_ref[...]
    p2y = p2y_ref[...]
    g0c = g0c_ref[...]
    g = jnp.tanh(_dot(x2, g0a_ref[...]) + p2x * g0c[0:1, :]
                 + p2y * g0c[1:2, :] + g0_ref[...])
    gg = _dot(g, g1_ref[...]) + g1b_ref[...]
    out_ref[...] = jnp.concatenate(
        [jnp.max(gg[i * s2:(i + 1) * s2, :], axis=0, keepdims=True)
         for i in range(b)], axis=0)


def kernel(pos, zones_ids, W0, b0, W1, b1, A0, a0, A1, a1, C0, c0, C1, c1,
           G0, g0, G1, g1):
    f32 = jnp.float32
    bb, nn, _ = pos.shape
    s1 = nn // 4
    s2 = s1 // 4
    st1 = 16
    r1sq = 0.2 * 0.2
    r2sq = 0.4 * 0.4

    posx = pos[:, :, 0]
    posy = pos[:, :, 1]

    p1x, p1y, p2x, p2y = pl.pallas_call(
        functools.partial(_fps_body, s1=s1, s2=s2),
        out_shape=(jax.ShapeDtypeStruct((bb, s1), f32),
                   jax.ShapeDtypeStruct((bb, s1), f32),
                   jax.ShapeDtypeStruct((bb, s2), f32),
                   jax.ShapeDtypeStruct((bb, s2), f32)),
    )(posx, posy)

    rt = 1024
    local2, t1f = pl.pallas_call(
        _feat_body,
        grid=(bb * nn // rt,),
        in_specs=[
            pl.BlockSpec((rt, 2), lambda r: (r, 0)),
            pl.BlockSpec((rt, 8), lambda r: (r, 0)),
            pl.BlockSpec((2, 32), lambda r: (0, 0)),
            pl.BlockSpec((1, 32), lambda r: (0, 0)),
            pl.BlockSpec((32, 32), lambda r: (0, 0)),
            pl.BlockSpec((1, 32), lambda r: (0, 0)),
            pl.BlockSpec((32, 64), lambda r: (0, 0)),
            pl.BlockSpec((8, 64), lambda r: (0, 0)),
            pl.BlockSpec((2, 64), lambda r: (0, 0)),
            pl.BlockSpec((1, 64), lambda r: (0, 0)),
        ],
        out_specs=(pl.BlockSpec((rt, 32), lambda r: (r, 0)),
                   pl.BlockSpec((rt, 64), lambda r: (r, 0))),
        out_shape=(jax.ShapeDtypeStruct((bb * nn, 32), f32),
                   jax.ShapeDtypeStruct((bb * nn, 64), f32)),
        compiler_params=_PAR1,
    )(pos.reshape(bb * nn, 2), zones_ids.reshape(bb * nn, 8),
      W0, b0.reshape(1, -1), W1, b1.reshape(1, -1),
      A0[0:32], A0[32:40], A0[40:42], a0.reshape(1, -1))

    pen1 = pl.pallas_call(
        functools.partial(_pen_body, r2=r1sq, kmax=_KMAX, st=st1),
        grid=(bb,),
        in_specs=[
            pl.BlockSpec((1, nn, 1), lambda b: (b, 0, 0)),
            pl.BlockSpec((1, nn, 1), lambda b: (b, 0, 0)),
            pl.BlockSpec((1, 1, s1), lambda b: (b, 0, 0)),
            pl.BlockSpec((1, 1, s1), lambda b: (b, 0, 0)),
        ],
        out_specs=pl.BlockSpec((1, s1 // st1, nn, st1), lambda b: (b, 0, 0, 0)),
        out_shape=jax.ShapeDtypeStruct((bb, s1 // st1, nn, st1), f32),
        compiler_params=_PAR1,
    )(posx[:, :, None], posy[:, :, None], p1x[:, None, :], p1y[:, None, :])

    t1 = t1f.reshape(bb, nn, 64)
    x1 = pl.pallas_call(
        _sa1_body,
        grid=(bb, s1 // st1),
        in_specs=[
            pl.BlockSpec((1, nn, 64), lambda b, s: (b, 0, 0)),
            pl.BlockSpec((1, 1, nn, st1), lambda b, s: (b, s, 0, 0)),
            pl.BlockSpec((1, st1, 1), lambda b, s: (b, s, 0)),
            pl.BlockSpec((1, st1, 1), lambda b, s: (b, s, 0)),
            pl.BlockSpec((2, 64), lambda b, s: (0, 0)),
            pl.BlockSpec((64, 64), lambda b, s: (0, 0)),
            pl.BlockSpec((1, 64), lambda b, s: (0, 0)),
        ],
        out_specs=pl.BlockSpec((1, st1, 64), lambda b, s: (b, s, 0)),
        out_shape=jax.ShapeDtypeStruct((bb, s1, 64), f32),
        compiler_params=_PAR2,
    )(t1, pen1, p1x[:, :, None], p1y[:, :, None],
      A0[40:42], A1, a1.reshape(1, -1))

    x2 = pl.pallas_call(
        functools.partial(_sa2_body, r2=r2sq, kmax=_KMAX),
        grid=(bb,),
        in_specs=[
            pl.BlockSpec((1, s1, 64), lambda b: (b, 0, 0)),
            pl.BlockSpec((1, s1, 1), lambda b: (b, 0, 0)),
            pl.BlockSpec((1, s1, 1), lambda b: (b, 0, 0)),
            pl.BlockSpec((1, 1, s2), lambda b: (b, 0, 0)),
            pl.BlockSpec((1, 1, s2), lambda b: (b, 0, 0)),
            pl.BlockSpec((1, s2, 1), lambda b: (b, 0, 0)),
            pl.BlockSpec((1, s2, 1), lambda b: (b, 0, 0)),
            pl.BlockSpec((64, 128), lambda b: (0, 0)),
            pl.BlockSpec((2, 128), lambda b: (0, 0)),
            pl.BlockSpec((1, 128), lambda b: (0, 0)),
            pl.BlockSpec((128, 128), lambda b: (0, 0)),
            pl.BlockSpec((1, 128), lambda b: (0, 0)),
        ],
        out_specs=pl.BlockSpec((1, s2, 128), lambda b: (b, 0, 0)),
        out_shape=jax.ShapeDtypeStruct((bb, s2, 128), f32),
        compiler_params=_PAR1,
    )(x1, p1x[:, :, None], p1y[:, :, None], p2x[:, None, :],
      p2y[:, None, :], p2x[:, :, None], p2y[:, :, None],
      C0[0:64], C0[64:66], c0.reshape(1, -1), C1, c1.reshape(1, -1))

    glob = pl.pallas_call(
        functools.partial(_glob_body, b=bb, s2=s2),
        out_shape=jax.ShapeDtypeStruct((bb, 256), f32),
    )(x2.reshape(bb * s2, 128), p2x.reshape(bb * s2, 1),
      p2y.reshape(bb * s2, 1), G0[0:128], G0[128:130], g0.reshape(1, -1),
      G1, g1.reshape(1, -1))

    return local2.reshape(bb, nn, 32), glob


# speedup vs baseline: 11.8131x; 1.0265x over previous
import functools

import jax
import jax.numpy as jnp
from jax.experimental import pallas as pl
from jax.experimental.pallas import tpu as pltpu

_PAR1 = pltpu.CompilerParams(dimension_semantics=("parallel",))
_PAR2 = pltpu.CompilerParams(dimension_semantics=("parallel", "parallel"))

_INT_INF = 0x7F800000
_KMAX = 128


def _fps_stage(px, py, n_sample):
    b, np_ = px.shape
    j_n = jax.lax.broadcasted_iota(jnp.int32, (b, np_), 1)
    j_s = jax.lax.broadcasted_iota(jnp.int32, (b, n_sample), 1)

    def gather_col(x, sel):
        return jnp.sum(jnp.where(j_n == sel, x, jnp.float32(0.0)), axis=1,
                       keepdims=True)

    def body(i, st):
        sel, dmin, ox, oy = st
        lx = gather_col(px, sel)
        ly = gather_col(py, sel)
        ox = jnp.where(j_s == i - 1, lx, ox)
        oy = jnp.where(j_s == i - 1, ly, oy)
        d = (px - lx) ** 2 + (py - ly) ** 2
        dmin = jnp.minimum(dmin, d)
        m = jnp.max(dmin, axis=1, keepdims=True)
        nxt = jnp.min(jnp.where(dmin == m, j_n, jnp.int32(np_)), axis=1,
                      keepdims=True)
        return nxt, dmin, ox, oy

    init = (jnp.zeros((b, 1), jnp.int32),
            jnp.full((b, np_), jnp.inf, jnp.float32),
            jnp.zeros((b, n_sample), jnp.float32),
            jnp.zeros((b, n_sample), jnp.float32))
    sel, _, ox, oy = jax.lax.fori_loop(1, n_sample, body, init)
    lx = gather_col(px, sel)
    ly = gather_col(py, sel)
    ox = jnp.where(j_s == n_sample - 1, lx, ox)
    oy = jnp.where(j_s == n_sample - 1, ly, oy)
    return ox, oy


def _fps_body(px_ref, py_ref, p1x_ref, p1y_ref, p2x_ref, p2y_ref, *, s1, s2):
    px = px_ref[...]
    py = py_ref[...]
    p1x, p1y = _fps_stage(px, py, s1)
    p2x, p2y = _fps_stage(p1x, p1y, s2)
    p1x_ref[...] = p1x
    p1y_ref[...] = p1y
    p2x_ref[...] = p2x
    p2y_ref[...] = p2y


def _dot(a, b):
    return jnp.dot(a, b, preferred_element_type=jnp.float32)


def _feat_body(p_ref, z_ref, w0_ref, b0_ref, w1_ref, b1_ref, a0a_ref, a0b_ref,
               a0c_ref, a0_ref, local_ref, t1_ref):
    p = p_ref[...]
    z = z_ref[...]
    h = jnp.tanh(_dot(p, w0_ref[...]) + b0_ref[...])
    loc = jnp.tanh(_dot(h, w1_ref[...]) + b1_ref[...])
    local_ref[...] = loc
    t1_ref[...] = (_dot(loc, a0a_ref[...]) + _dot(z, a0b_ref[...])
                   + _dot(p, a0c_ref[...]) + a0_ref[...])


def _select_mask_t(d2t, r2, kmax):
    s = d2t.shape[1]
    inrad = d2t <= jnp.float32(r2)
    u = jax.lax.bitcast_convert_type(d2t, jnp.int32)
    um = jnp.where(inrad, u, jnp.int32(_INT_INF))

    def bs(_, st):
        lo, hi = st
        mid = lo + (hi - lo) // 2
        cnt = jnp.sum((um <= mid).astype(jnp.int32), axis=0, keepdims=True)
        ge = cnt >= kmax
        return jnp.where(ge, lo, mid), jnp.where(ge, mid, hi)

    lo0 = jnp.full((1, s), -1, jnp.int32)
    hi0 = jnp.full((1, s), _INT_INF, jnp.int32)
    _, hi = jax.lax.fori_loop(0, 32, bs, (lo0, hi0))
    return jnp.logical_and(inrad, um <= hi)


def _conv_max(t, ucen, pent, w2, b2):
    s = ucen.shape[0]
    np_ = t.shape[0]
    pre = jnp.concatenate([t - ucen[i:i + 1, :] for i in range(s)], axis=0)
    h2 = jnp.dot(jnp.tanh(pre), w2,
                 preferred_element_type=jnp.float32) + b2
    pcol = jnp.concatenate([pent[:, i:i + 1] for i in range(s)], axis=0)
    h2 = h2 + pcol
    return jnp.concatenate(
        [jnp.max(h2[i * np_:(i + 1) * np_, :], axis=0, keepdims=True)
         for i in range(s)], axis=0)


def _pen_body(pxc_ref, pyc_ref, cxr_ref, cyr_ref, out_ref, *, r2, kmax, st):
    pxc = pxc_ref[0]
    pyc = pyc_ref[0]
    cxr = cxr_ref[0]
    cyr = cyr_ref[0]
    d2t = (cxr - pxc) ** 2 + (cyr - pyc) ** 2
    maskt = _select_mask_t(d2t, r2, kmax)
    pent = jnp.where(maskt, jnp.float32(0.0), -jnp.inf)
    s = d2t.shape[1]
    for k in range(s // st):
        out_ref[0, k] = pent[:, k * st:(k + 1) * st]


def _sa1_body(t_ref, pen_ref, cxc_ref, cyc_ref, wc_ref, w2_ref, b2_ref,
              out_ref):
    t = t_ref[0]
    pent = pen_ref[0, 0]
    cxc = cxc_ref[0]
    cyc = cyc_ref[0]
    ucen = cxc * wc_ref[0:1, :] + cyc * wc_ref[1:2, :]
    out_ref[...] = _conv_max(t, ucen, pent, w2_ref[...], b2_ref[...])[None]


def _sa2_body(x1_ref, p1xc_ref, p1yc_ref, p2xr_ref, p2yr_ref, p2xc_ref,
              p2yc_ref, c0a_ref, c0c_ref, c0_ref, c1_ref, c1b_ref, out_ref,
              *, r2, kmax):
    x1 = x1_ref[0]
    p1xc = p1xc_ref[0]
    p1yc = p1yc_ref[0]
    p2xr = p2xr_ref[0]
    p2yr = p2yr_ref[0]
    p2xc = p2xc_ref[0]
    p2yc = p2yc_ref[0]
    c0c = c0c_ref[...]
    t2 = (_dot(x1, c0a_ref[...]) + p1xc * c0c[0:1, :] + p1yc * c0c[1:2, :]
          + c0_ref[...])
    d2t = (p2xr - p1xc) ** 2 + (p2yr - p1yc) ** 2
    maskt = _select_mask_t(d2t, r2, kmax)
    pent = jnp.where(maskt, jnp.float32(0.0), -jnp.inf)
    ucen = p2xc * c0c[0:1, :] + p2yc * c0c[1:2, :]
    out_ref[...] = _conv_max(t2, ucen, pent, c1_ref[...],
                             c1b_ref[...])[None]


def _glob_body(x2_ref, p2x_ref, p2y_ref, g0a_ref, g0c_ref, g0_ref, g1_ref,
               g1b_ref, out_ref, *, b, s2):
    x2 = x2_ref[...]
    p2x = p2x_ref[...]
    p2y = p2y_ref[...]
    g0c = g0c_ref[...]
    g = jnp.tanh(_dot(x2, g0a_ref[...]) + p2x * g0c[0:1, :]
                 + p2y * g0c[1:2, :] + g0_ref[...])
    gg = _dot(g, g1_ref[...]) + g1b_ref[...]
    out_ref[...] = jnp.concatenate(
        [jnp.max(gg[i * s2:(i + 1) * s2, :], axis=0, keepdims=True)
         for i in range(b)], axis=0)


def kernel(pos, zones_ids, W0, b0, W1, b1, A0, a0, A1, a1, C0, c0, C1, c1,
           G0, g0, G1, g1):
    f32 = jnp.float32
    bb, nn, _ = pos.shape
    s1 = nn // 4
    s2 = s1 // 4
    st1 = 16
    r1sq = 0.2 * 0.2
    r2sq = 0.4 * 0.4

    posx = pos[:, :, 0]
    posy = pos[:, :, 1]

    p1x, p1y, p2x, p2y = pl.pallas_call(
        functools.partial(_fps_body, s1=s1, s2=s2),
        out_shape=(jax.ShapeDtypeStruct((bb, s1), f32),
                   jax.ShapeDtypeStruct((bb, s1), f32),
                   jax.ShapeDtypeStruct((bb, s2), f32),
                   jax.ShapeDtypeStruct((bb, s2), f32)),
    )(posx, posy)

    rt = 1024
    local2, t1f = pl.pallas_call(
        _feat_body,
        grid=(bb * nn // rt,),
        in_specs=[
            pl.BlockSpec((rt, 2), lambda r: (r, 0)),
            pl.BlockSpec((rt, 8), lambda r: (r, 0)),
            pl.BlockSpec((2, 32), lambda r: (0, 0)),
            pl.BlockSpec((1, 32), lambda r: (0, 0)),
            pl.BlockSpec((32, 32), lambda r: (0, 0)),
            pl.BlockSpec((1, 32), lambda r: (0, 0)),
            pl.BlockSpec((32, 64), lambda r: (0, 0)),
            pl.BlockSpec((8, 64), lambda r: (0, 0)),
            pl.BlockSpec((2, 64), lambda r: (0, 0)),
            pl.BlockSpec((1, 64), lambda r: (0, 0)),
        ],
        out_specs=(pl.BlockSpec((rt, 32), lambda r: (r, 0)),
                   pl.BlockSpec((rt, 64), lambda r: (r, 0))),
        out_shape=(jax.ShapeDtypeStruct((bb * nn, 32), f32),
                   jax.ShapeDtypeStruct((bb * nn, 64), f32)),
        compiler_params=_PAR1,
    )(pos.reshape(bb * nn, 2), zones_ids.reshape(bb * nn, 8),
      W0, b0.reshape(1, -1), W1, b1.reshape(1, -1),
      A0[0:32], A0[32:40], A0[40:42], a0.reshape(1, -1))

    pen1 = pl.pallas_call(
        functools.partial(_pen_body, r2=r1sq, kmax=_KMAX, st=st1),
        grid=(bb,),
        in_specs=[
            pl.BlockSpec((1, nn, 1), lambda b: (b, 0, 0)),
            pl.BlockSpec((1, nn, 1), lambda b: (b, 0, 0)),
            pl.BlockSpec((1, 1, s1), lambda b: (b, 0, 0)),
            pl.BlockSpec((1, 1, s1), lambda b: (b, 0, 0)),
        ],
        out_specs=pl.BlockSpec((1, s1 // st1, nn, st1), lambda b: (b, 0, 0, 0)),
        out_shape=jax.ShapeDtypeStruct((bb, s1 // st1, nn, st1), f32),
        compiler_params=_PAR1,
    )(posx[:, :, None], posy[:, :, None], p1x[:, None, :], p1y[:, None, :])

    t1 = t1f.reshape(bb, nn, 64)
    x1 = pl.pallas_call(
        _sa1_body,
        grid=(bb, s1 // st1),
        in_specs=[
            pl.BlockSpec((1, nn, 64), lambda b, s: (b, 0, 0)),
            pl.BlockSpec((1, 1, nn, st1), lambda b, s: (b, s, 0, 0)),
            pl.BlockSpec((1, st1, 1), lambda b, s: (b, s, 0)),
            pl.BlockSpec((1, st1, 1), lambda b, s: (b, s, 0)),
            pl.BlockSpec((2, 64), lambda b, s: (0, 0)),
            pl.BlockSpec((64, 64), lambda b, s: (0, 0)),
            pl.BlockSpec((1, 64), lambda b, s: (0, 0)),
        ],
        out_specs=pl.BlockSpec((1, st1, 64), lambda b, s: (b, s, 0)),
        out_shape=jax.ShapeDtypeStruct((bb, s1, 64), f32),
        compiler_params=_PAR2,
    )(t1, pen1, p1x[:, :, None], p1y[:, :, None],
      A0[40:42], A1, a1.reshape(1, -1))

    x2 = pl.pallas_call(
        functools.partial(_sa2_body, r2=r2sq, kmax=_KMAX),
        grid=(bb,),
        in_specs=[
            pl.BlockSpec((1, s1, 64), lambda b: (b, 0, 0)),
            pl.BlockSpec((1, s1, 1), lambda b: (b, 0, 0)),
            pl.BlockSpec((1, s1, 1), lambda b: (b, 0, 0)),
            pl.BlockSpec((1, 1, s2), lambda b: (b, 0, 0)),
            pl.BlockSpec((1, 1, s2), lambda b: (b, 0, 0)),
            pl.BlockSpec((1, s2, 1), lambda b: (b, 0, 0)),
            pl.BlockSpec((1, s2, 1), lambda b: (b, 0, 0)),
            pl.BlockSpec((64, 128), lambda b: (0, 0)),
            pl.BlockSpec((2, 128), lambda b: (0, 0)),
            pl.BlockSpec((1, 128), lambda b: (0, 0)),
            pl.BlockSpec((128, 128), lambda b: (0, 0)),
            pl.BlockSpec((1, 128), lambda b: (0, 0)),
        ],
        out_specs=pl.BlockSpec((1, s2, 128), lambda b: (b, 0, 0)),
        out_shape=jax.ShapeDtypeStruct((bb, s2, 128), f32),
        compiler_params=_PAR1,
    )(x1, p1x[:, :, None], p1y[:, :, None], p2x[:, None, :],
      p2y[:, None, :], p2x[:, :, None], p2y[:, :, None],
      C0[0:64], C0[64:66], c0.reshape(1, -1), C1, c1.reshape(1, -1))

    glob = pl.pallas_call(
        functools.partial(_glob_body, b=bb, s2=s2),
        out_shape=jax.ShapeDtypeStruct((bb, 256), f32),
    )(x2.reshape(bb * s2, 128), p2x.reshape(bb * s2, 1),
      p2y.reshape(bb * s2, 1), G0[0:128], G0[128:130], g0.reshape(1, -1),
      G1, g1.reshape(1, -1))

    return local2.reshape(bb, nn, 32), glob
